# baseline (speedup 1.0000x reference)
import functools

import jax
import jax.numpy as jnp
from jax import lax
from jax.experimental import pallas as pl
from jax.experimental.pallas import tpu as pltpu
from jax.experimental.pallas import tpu_sc as plsc

B, T, K = 1024, 200, 128
H = K + 5
NC, NS, L = 2, 16, 16
NW = NC * NS
ROWS = B // NW
NBATCH = ROWS // L

_LN2 = 0.6931471805599453


def _log_ge1(x):
    bits = lax.bitcast_convert_type(x, jnp.int32)
    e = (bits >> 23) - 127
    m = lax.bitcast_convert_type((bits & 0x007FFFFF) | 0x3F800000, jnp.float32)
    z = (m - 1.0) / (m + 1.0)
    z2 = z * z
    logm = 2.0 * z * (1.0 + z2 * (1.0 / 3.0 + z2 * (0.2 + z2 * (1.0 / 7.0))))
    return e.astype(jnp.float32) * _LN2 + logm


def _make_agg():
    mesh = plsc.VectorSubcoreMesh(core_axis_name="c", subcore_axis_name="s")

    @functools.partial(
        pl.kernel,
        out_type=jax.ShapeDtypeStruct((B, H), jnp.float32),
        mesh=mesh,
        compiler_params=pltpu.CompilerParams(needs_layout_passes=False),
        scratch_types=[
            pltpu.VMEM((ROWS, T), jnp.float32),
            pltpu.VMEM((ROWS, T), jnp.int32),
            pltpu.VMEM((ROWS,), jnp.int32),
            pltpu.VMEM((ROWS, H), jnp.float32),
            pltpu.SemaphoreType.DMA,
            pltpu.SemaphoreType.DMA,
            pltpu.SemaphoreType.DMA,
        ],
    )
    def agg(amt_hbm, mcc_hbm, sl_hbm, out_hbm, amt_v, mcc_v, sl_v, outb_v,
            sem_a, sem_m, sem_s):
        wid = lax.axis_index("s") * NC + lax.axis_index("c")
        base = wid * ROWS
        h_a = pltpu.async_copy(amt_hbm.at[pl.ds(base, ROWS)], amt_v, sem_a)
        h_m = pltpu.async_copy(mcc_hbm.at[pl.ds(base, ROWS)], mcc_v, sem_m)
        h_s = pltpu.async_copy(sl_hbm.at[pl.ds(base, ROWS)], sl_v, sem_s)

        lanes = lax.iota(jnp.int32, L)
        ones = jnp.full((L,), 1.0, jnp.float32)
        zeros = jnp.zeros((L,), jnp.float32)
        rows = [lanes + nb * L for nb in range(NBATCH)]

        @plsc.parallel_loop(4, H - 1, 1, unroll=8)
        def _zero(c):
            for nb in range(NBATCH):
                plsc.store_scatter(outb_v, [rows[nb], c + lanes * 0], zeros)

        h_a.wait()
        h_m.wait()
        h_s.wait()

        init = tuple((zeros, zeros, zeros) for _ in range(NBATCH))

        @plsc.parallel_loop(0, T, 1, unroll=4, carry=init)
        def sums(t, carry):
            tcol = lanes * 0 + t
            new = []
            for nb in range(NBATCH):
                sp, sn, st = carry[nb]
                a = plsc.load_gather(amt_v, [rows[nb], tcol])
                code = plsc.load_gather(mcc_v, [rows[nb], tcol])
                plsc.addupdate_scatter(outb_v, [rows[nb], code + 4], ones)
                new.append((sp + jnp.maximum(a, 0.0),
                            sn + jnp.minimum(a, 0.0),
                            st + a))
            return tuple(new)

        for nb in range(NBATCH):
            sp, sn, st = sums[nb]
            r = rows[nb]
            zcol = lanes * 0
            sl_f = sl_v[pl.ds(nb * L, L)].astype(jnp.float32)
            plsc.store_scatter(outb_v, [r, zcol], sl_f)
            plsc.store_scatter(outb_v, [r, zcol + 1], _log_ge1(sp + 1.0))
            plsc.store_scatter(outb_v, [r, zcol + 2], -_log_ge1(1.0 - sn))
            plsc.store_scatter(outb_v, [r, zcol + 3], st / (sl_f + 1e-9))
            plsc.store_scatter(outb_v, [r, zcol + 4], zeros)

        @plsc.parallel_loop(0, K - 1, 1, unroll=8,
                            carry=tuple(zeros for _ in range(NBATCH)))
        def distincts(k, accs):
            kcol = lanes * 0 + (k + 5)
            return tuple(
                accs[nb] + jnp.where(
                    plsc.load_gather(outb_v, [rows[nb], kcol]) > 0.0,
                    1.0, 0.0)
                for nb in range(NBATCH))

        for nb in range(NBATCH):
            plsc.store_scatter(outb_v, [rows[nb], lanes * 0 + (H - 1)],
                               distincts[nb])

        pltpu.sync_copy(outb_v, out_hbm.at[pl.ds(base, ROWS)])

    return agg


_agg = _make_agg()


def kernel(amount, mcc_code, seq_lens, ohe_mcc_code):
    del ohe_mcc_code
    return _agg(amount, mcc_code, seq_lens)


# speedup vs baseline: 1.0016x; 1.0016x over previous
import functools

import jax
import jax.numpy as jnp
from jax import lax
from jax.experimental import pallas as pl
from jax.experimental.pallas import tpu as pltpu
from jax.experimental.pallas import tpu_sc as plsc

B, T, K = 1024, 200, 128
H = K + 5
NC, NS, L = 2, 16, 16
NW = NC * NS
ROWS = B // NW
NBATCH = ROWS // L

_LN2 = 0.6931471805599453


def _log_ge1(x):
    bits = lax.bitcast_convert_type(x, jnp.int32)
    e = (bits >> 23) - 127
    m = lax.bitcast_convert_type((bits & 0x007FFFFF) | 0x3F800000, jnp.float32)
    z = (m - 1.0) / (m + 1.0)
    z2 = z * z
    logm = 2.0 * z * (1.0 + z2 * (1.0 / 3.0 + z2 * (0.2 + z2 * (1.0 / 7.0))))
    return e.astype(jnp.float32) * _LN2 + logm


def _make_agg():
    mesh = plsc.VectorSubcoreMesh(core_axis_name="c", subcore_axis_name="s")

    @functools.partial(
        pl.kernel,
        out_type=jax.ShapeDtypeStruct((B, H), jnp.float32),
        mesh=mesh,
        compiler_params=pltpu.CompilerParams(needs_layout_passes=False,
                                             use_tc_tiling_on_sc=True),
        scratch_types=[
            pltpu.VMEM((ROWS, T), jnp.float32),
            pltpu.VMEM((ROWS, T), jnp.int32),
            pltpu.VMEM((ROWS,), jnp.int32),
            pltpu.VMEM((ROWS, H), jnp.float32),
            pltpu.SemaphoreType.DMA,
            pltpu.SemaphoreType.DMA,
            pltpu.SemaphoreType.DMA,
        ],
    )
    def agg(amt_hbm, mcc_hbm, sl_hbm, out_hbm, amt_v, mcc_v, sl_v, outb_v,
            sem_a, sem_m, sem_s):
        wid = lax.axis_index("s") * NC + lax.axis_index("c")
        base = wid * ROWS
        h_a = pltpu.async_copy(amt_hbm.at[pl.ds(base, ROWS)], amt_v, sem_a)
        h_m = pltpu.async_copy(mcc_hbm.at[pl.ds(base, ROWS)], mcc_v, sem_m)
        h_s = pltpu.async_copy(sl_hbm.at[pl.ds(base, ROWS)], sl_v, sem_s)

        lanes = lax.iota(jnp.int32, L)
        ones = jnp.full((L,), 1.0, jnp.float32)
        zeros = jnp.zeros((L,), jnp.float32)
        rows = [lanes + nb * L for nb in range(NBATCH)]

        @plsc.parallel_loop(4, H - 1, 1, unroll=8)
        def _zero(c):
            for nb in range(NBATCH):
                plsc.store_scatter(outb_v, [rows[nb], c + lanes * 0], zeros)

        h_a.wait()
        h_m.wait()
        h_s.wait()

        init = tuple((zeros, zeros, zeros) for _ in range(NBATCH))

        @plsc.parallel_loop(0, T, 1, unroll=4, carry=init)
        def sums(t, carry):
            tcol = lanes * 0 + t
            new = []
            for nb in range(NBATCH):
                sp, sn, st = carry[nb]
                a = plsc.load_gather(amt_v, [rows[nb], tcol])
                code = plsc.load_gather(mcc_v, [rows[nb], tcol])
                plsc.addupdate_scatter(outb_v, [rows[nb], code + 4], ones)
                new.append((sp + jnp.maximum(a, 0.0),
                            sn + jnp.minimum(a, 0.0),
                            st + a))
            return tuple(new)

        for nb in range(NBATCH):
            sp, sn, st = sums[nb]
            r = rows[nb]
            zcol = lanes * 0
            sl_f = sl_v[pl.ds(nb * L, L)].astype(jnp.float32)
            plsc.store_scatter(outb_v, [r, zcol], sl_f)
            plsc.store_scatter(outb_v, [r, zcol + 1], _log_ge1(sp + 1.0))
            plsc.store_scatter(outb_v, [r, zcol + 2], -_log_ge1(1.0 - sn))
            plsc.store_scatter(outb_v, [r, zcol + 3], st / (sl_f + 1e-9))
            plsc.store_scatter(outb_v, [r, zcol + 4], zeros)

        @plsc.parallel_loop(0, K - 1, 1, unroll=8,
                            carry=tuple(zeros for _ in range(NBATCH)))
        def distincts(k, accs):
            kcol = lanes * 0 + (k + 5)
            return tuple(
                accs[nb] + jnp.where(
                    plsc.load_gather(outb_v, [rows[nb], kcol]) > 0.0,
                    1.0, 0.0)
                for nb in range(NBATCH))

        for nb in range(NBATCH):
            plsc.store_scatter(outb_v, [rows[nb], lanes * 0 + (H - 1)],
                               distincts[nb])

        pltpu.sync_copy(outb_v, out_hbm.at[pl.ds(base, ROWS)])

    return agg


_agg = _make_agg()


def kernel(amount, mcc_code, seq_lens, ohe_mcc_code):
    del ohe_mcc_code
    return _agg(amount, mcc_code, seq_lens)


# speedup vs baseline: 1.0728x; 1.0711x over previous
import functools

import jax
import jax.numpy as jnp
from jax import lax
from jax.experimental import pallas as pl
from jax.experimental.pallas import tpu as pltpu
from jax.experimental.pallas import tpu_sc as plsc

B, T, K = 1024, 200, 128
H = K + 5
NC, NS, L = 2, 16, 16
NW = NC * NS
ROWS = B // NW
NBATCH = ROWS // L

_LN2 = 0.6931471805599453


def _log_ge1(x):
    bits = lax.bitcast_convert_type(x, jnp.int32)
    e = (bits >> 23) - 127
    m = lax.bitcast_convert_type((bits & 0x007FFFFF) | 0x3F800000, jnp.float32)
    z = (m - 1.0) / (m + 1.0)
    z2 = z * z
    logm = 2.0 * z * (1.0 + z2 * (1.0 / 3.0 + z2 * (0.2 + z2 * (1.0 / 7.0))))
    return e.astype(jnp.float32) * _LN2 + logm


def _make_agg():
    mesh = plsc.VectorSubcoreMesh(core_axis_name="c", subcore_axis_name="s")

    @functools.partial(
        pl.kernel,
        out_type=jax.ShapeDtypeStruct((B, H), jnp.float32),
        mesh=mesh,
        compiler_params=pltpu.CompilerParams(needs_layout_passes=False),
        scratch_types=[
            pltpu.VMEM((ROWS * T,), jnp.float32),
            pltpu.VMEM((ROWS * T,), jnp.int32),
            pltpu.VMEM((ROWS,), jnp.int32),
            pltpu.VMEM((ROWS, H), jnp.float32),
            pltpu.SemaphoreType.DMA,
            pltpu.SemaphoreType.DMA,
            pltpu.SemaphoreType.DMA,
        ],
    )
    def agg(amt_hbm, mcc_hbm, sl_hbm, out_hbm, amt_v, mcc_v, sl_v, outb_v,
            sem_a, sem_m, sem_s):
        wid = lax.axis_index("s") * NC + lax.axis_index("c")
        base = wid * ROWS
        h_a = pltpu.async_copy(amt_hbm.at[pl.ds(base * T, ROWS * T)],
                               amt_v, sem_a)
        h_m = pltpu.async_copy(mcc_hbm.at[pl.ds(base * T, ROWS * T)],
                               mcc_v, sem_m)
        h_s = pltpu.async_copy(sl_hbm.at[pl.ds(base, ROWS)], sl_v, sem_s)

        lanes = lax.iota(jnp.int32, L)
        ones = jnp.full((L,), 1.0, jnp.float32)
        zeros = jnp.zeros((L,), jnp.float32)
        rows = [lanes + nb * L for nb in range(NBATCH)]
        rowsT = [(lanes + nb * L) * T for nb in range(NBATCH)]

        @plsc.parallel_loop(4, H - 1, 1, unroll=8)
        def _zero(c):
            for nb in range(NBATCH):
                plsc.store_scatter(outb_v, [rows[nb], c + lanes * 0], zeros)

        h_a.wait()
        h_m.wait()
        h_s.wait()

        init = tuple((zeros, zeros, zeros) for _ in range(NBATCH))

        @plsc.parallel_loop(0, T, 1, unroll=4, carry=init)
        def sums(t, carry):
            new = []
            for nb in range(NBATCH):
                sp, sn, st = carry[nb]
                idx = rowsT[nb] + t
                a = plsc.load_gather(amt_v, [idx])
                code = plsc.load_gather(mcc_v, [idx])
                plsc.addupdate_scatter(outb_v, [rows[nb], code + 4], ones)
                new.append((sp + jnp.maximum(a, 0.0),
                            sn + jnp.minimum(a, 0.0),
                            st + a))
            return tuple(new)

        for nb in range(NBATCH):
            sp, sn, st = sums[nb]
            r = rows[nb]
            zcol = lanes * 0
            sl_f = sl_v[pl.ds(nb * L, L)].astype(jnp.float32)
            plsc.store_scatter(outb_v, [r, zcol], sl_f)
            plsc.store_scatter(outb_v, [r, zcol + 1], _log_ge1(sp + 1.0))
            plsc.store_scatter(outb_v, [r, zcol + 2], -_log_ge1(1.0 - sn))
            plsc.store_scatter(outb_v, [r, zcol + 3], st / (sl_f + 1e-9))
            plsc.store_scatter(outb_v, [r, zcol + 4], zeros)

        @plsc.parallel_loop(0, K - 1, 1, unroll=8,
                            carry=tuple(zeros for _ in range(NBATCH)))
        def distincts(k, accs):
            kcol = lanes * 0 + (k + 5)
            return tuple(
                accs[nb] + jnp.where(
                    plsc.load_gather(outb_v, [rows[nb], kcol]) > 0.0,
                    1.0, 0.0)
                for nb in range(NBATCH))

        for nb in range(NBATCH):
            plsc.store_scatter(outb_v, [rows[nb], lanes * 0 + (H - 1)],
                               distincts[nb])

        pltpu.sync_copy(outb_v, out_hbm.at[pl.ds(base, ROWS)])

    return agg


_agg = _make_agg()


def kernel(amount, mcc_code, seq_lens, ohe_mcc_code):
    del ohe_mcc_code
    return _agg(amount.reshape(-1), mcc_code.reshape(-1), seq_lens)


# speedup vs baseline: 1.1535x; 1.0753x over previous
import functools

import jax
import jax.numpy as jnp
from jax import lax
from jax.experimental import pallas as pl
from jax.experimental.pallas import tpu as pltpu
from jax.experimental.pallas import tpu_sc as plsc

B, T, K = 1024, 200, 128
H = K + 5
NC, NS, L = 2, 16, 16
NW = NC * NS
ROWS = B // NW
NBATCH = ROWS // L

_LN2 = 0.6931471805599453


def _log_ge1(x):
    bits = lax.bitcast_convert_type(x, jnp.int32)
    e = (bits >> 23) - 127
    m = lax.bitcast_convert_type((bits & 0x007FFFFF) | 0x3F800000, jnp.float32)
    z = (m - 1.0) / (m + 1.0)
    z2 = z * z
    logm = 2.0 * z * (1.0 + z2 * (1.0 / 3.0 + z2 * (0.2 + z2 * (1.0 / 7.0))))
    return e.astype(jnp.float32) * _LN2 + logm


def _make_agg():
    mesh = plsc.VectorSubcoreMesh(core_axis_name="c", subcore_axis_name="s")

    @functools.partial(
        pl.kernel,
        out_type=jax.ShapeDtypeStruct((B * H,), jnp.float32),
        mesh=mesh,
        compiler_params=pltpu.CompilerParams(needs_layout_passes=False),
        scratch_types=[
            pltpu.VMEM((ROWS * T,), jnp.int32),
            pltpu.VMEM((ROWS * T,), jnp.int32),
            pltpu.VMEM((ROWS,), jnp.int32),
            pltpu.VMEM((ROWS * H,), jnp.float32),
            pltpu.SemaphoreType.DMA,
            pltpu.SemaphoreType.DMA,
            pltpu.SemaphoreType.DMA,
        ],
    )
    def agg(big_hbm, sl_hbm, out_hbm, amt_v, mcc_v, sl_v, outb_v,
            sem_a, sem_m, sem_s):
        wid = lax.axis_index("s") * NC + lax.axis_index("c")
        h_a = pltpu.async_copy(big_hbm.at[pl.ds(wid * (ROWS * T), ROWS * T)],
                               amt_v, sem_a)
        h_m = pltpu.async_copy(
            big_hbm.at[pl.ds(B * T + wid * (ROWS * T), ROWS * T)],
            mcc_v, sem_m)
        h_s = pltpu.async_copy(sl_hbm.at[pl.ds(wid * ROWS, ROWS)], sl_v, sem_s)

        @plsc.parallel_loop(0, ROWS * H // L, 1, unroll=8)
        def _zero(i):
            outb_v[pl.ds(i * L, L)] = jnp.zeros((L,), jnp.float32)
        h_a.wait()
        h_m.wait()
        h_s.wait()

        lanes = lax.iota(jnp.int32, L)
        ones = jnp.full((L,), 1.0, jnp.float32)
        zeros = jnp.zeros((L,), jnp.float32)

        rowsT = [(lanes + nb * L) * T for nb in range(NBATCH)]
        rowsH = [(lanes + nb * L) * H for nb in range(NBATCH)]

        init = tuple((zeros, zeros, zeros) for _ in range(NBATCH))

        @plsc.parallel_loop(0, T, 1, unroll=4, carry=init)
        def sums(t, carry):
            new = []
            for nb in range(NBATCH):
                sp, sn, st = carry[nb]
                idx = rowsT[nb] + t
                a = plsc.bitcast(plsc.load_gather(amt_v, [idx]), jnp.float32)
                code = plsc.load_gather(mcc_v, [idx])
                plsc.addupdate_scatter(outb_v, [rowsH[nb] + 4 + code], ones)
                new.append((sp + jnp.maximum(a, 0.0),
                            sn + jnp.minimum(a, 0.0),
                            st + a))
            return tuple(new)

        for nb in range(NBATCH):
            sp, sn, st = sums[nb]
            rH = rowsH[nb]
            sl_f = sl_v[pl.ds(nb * L, L)].astype(jnp.float32)
            plsc.store_scatter(outb_v, [rH], sl_f)
            plsc.store_scatter(outb_v, [rH + 1], _log_ge1(sp + 1.0))
            plsc.store_scatter(outb_v, [rH + 2], -_log_ge1(1.0 - sn))
            plsc.store_scatter(outb_v, [rH + 3], st / (sl_f + 1e-9))
            plsc.store_scatter(outb_v, [rH + 4], zeros)

        @plsc.parallel_loop(0, K - 1, 1, unroll=8,
                            carry=tuple(zeros for _ in range(NBATCH)))
        def distincts(k, accs):
            return tuple(
                accs[nb] + jnp.where(
                    plsc.load_gather(outb_v, [rowsH[nb] + 5 + k]) > 0.0,
                    1.0, 0.0)
                for nb in range(NBATCH))

        for nb in range(NBATCH):
            plsc.store_scatter(outb_v, [rowsH[nb] + (H - 1)], distincts[nb])

        pltpu.sync_copy(outb_v, out_hbm.at[pl.ds(wid * (ROWS * H), ROWS * H)])

    return agg


_agg = _make_agg()


def kernel(amount, mcc_code, seq_lens, ohe_mcc_code):
    del ohe_mcc_code
    big = jnp.stack([lax.bitcast_convert_type(amount, jnp.int32), mcc_code])
    out_flat = _agg(big.reshape(-1), seq_lens)
    return out_flat.reshape(B, H)


# speedup vs baseline: 1.1542x; 1.0006x over previous
import functools

import jax
import jax.numpy as jnp
from jax import lax
from jax.experimental import pallas as pl
from jax.experimental.pallas import tpu as pltpu
from jax.experimental.pallas import tpu_sc as plsc

B, T, K = 1024, 200, 128
H = K + 5
NC, NS, L = 2, 16, 16
NW = NC * NS
ROWS = B // NW
NBATCH = ROWS // L

_LN2 = 0.6931471805599453


def _log_ge1(x):
    bits = lax.bitcast_convert_type(x, jnp.int32)
    e = (bits >> 23) - 127
    m = lax.bitcast_convert_type((bits & 0x007FFFFF) | 0x3F800000, jnp.float32)
    z = (m - 1.0) / (m + 1.0)
    z2 = z * z
    logm = 2.0 * z * (1.0 + z2 * (1.0 / 3.0 + z2 * (0.2 + z2 * (1.0 / 7.0))))
    return e.astype(jnp.float32) * _LN2 + logm


def _make_agg():
    mesh = plsc.VectorSubcoreMesh(core_axis_name="c", subcore_axis_name="s")

    @functools.partial(
        pl.kernel,
        out_type=jax.ShapeDtypeStruct((B * H,), jnp.float32),
        mesh=mesh,
        compiler_params=pltpu.CompilerParams(needs_layout_passes=False),
        scratch_types=[
            pltpu.VMEM((ROWS * T,), jnp.float32),
            pltpu.VMEM((ROWS * T,), jnp.int32),
            pltpu.VMEM((ROWS,), jnp.int32),
            pltpu.VMEM((ROWS * H,), jnp.float32),
            pltpu.SemaphoreType.DMA,
            pltpu.SemaphoreType.DMA,
            pltpu.SemaphoreType.DMA,
        ],
    )
    def agg(amt_hbm, mcc_hbm, sl_hbm, out_hbm, amt_v, mcc_v, sl_v, outb_v,
            sem_a, sem_m, sem_s):
        wid = lax.axis_index("s") * NC + lax.axis_index("c")
        h_a = pltpu.async_copy(amt_hbm.at[pl.ds(wid * (ROWS * T), ROWS * T)],
                               amt_v, sem_a)
        h_m = pltpu.async_copy(mcc_hbm.at[pl.ds(wid * (ROWS * T), ROWS * T)],
                               mcc_v, sem_m)
        h_s = pltpu.async_copy(sl_hbm.at[pl.ds(wid * ROWS, ROWS)], sl_v, sem_s)

        @plsc.parallel_loop(0, ROWS * H // L, 1, unroll=8)
        def _zero(i):
            outb_v[pl.ds(i * L, L)] = jnp.zeros((L,), jnp.float32)
        h_a.wait()
        h_m.wait()
        h_s.wait()

        lanes = lax.iota(jnp.int32, L)
        ones = jnp.full((L,), 1.0, jnp.float32)
        zeros = jnp.zeros((L,), jnp.float32)

        rowsT = [(lanes + nb * L) * T for nb in range(NBATCH)]
        rowsH = [(lanes + nb * L) * H for nb in range(NBATCH)]

        init = tuple((zeros, zeros, zeros) for _ in range(NBATCH))

        @plsc.parallel_loop(0, T, 1, unroll=8, carry=init)
        def sums(t, carry):
            new = []
            for nb in range(NBATCH):
                sp, sn, st = carry[nb]
                idx = rowsT[nb] + t
                a = plsc.load_gather(amt_v, [idx])
                code = plsc.load_gather(mcc_v, [idx])
                plsc.addupdate_scatter(outb_v, [rowsH[nb] + 4 + code], ones)
                new.append((sp + jnp.maximum(a, 0.0),
                            sn + jnp.minimum(a, 0.0),
                            st + a))
            return tuple(new)

        for nb in range(NBATCH):
            sp, sn, st = sums[nb]
            rH = rowsH[nb]
            sl_f = sl_v[pl.ds(nb * L, L)].astype(jnp.float32)
            plsc.store_scatter(outb_v, [rH], sl_f)
            plsc.store_scatter(outb_v, [rH + 1], _log_ge1(sp + 1.0))
            plsc.store_scatter(outb_v, [rH + 2], -_log_ge1(1.0 - sn))
            plsc.store_scatter(outb_v, [rH + 3], st / (sl_f + 1e-9))
            plsc.store_scatter(outb_v, [rH + 4], zeros)

        @plsc.parallel_loop(0, K - 1, 1, unroll=16,
                            carry=tuple(zeros for _ in range(NBATCH)))
        def distincts(k, accs):
            return tuple(
                accs[nb] + jnp.where(
                    plsc.load_gather(outb_v, [rowsH[nb] + 5 + k]) > 0.0,
                    1.0, 0.0)
                for nb in range(NBATCH))

        for nb in range(NBATCH):
            plsc.store_scatter(outb_v, [rowsH[nb] + (H - 1)], distincts[nb])

        pltpu.sync_copy(outb_v, out_hbm.at[pl.ds(wid * (ROWS * H), ROWS * H)])

    return agg


_agg = _make_agg()


def kernel(amount, mcc_code, seq_lens, ohe_mcc_code):
    del ohe_mcc_code
    out_flat = _agg(amount.reshape(-1), mcc_code.reshape(-1), seq_lens)
    return out_flat.reshape(B, H)


# speedup vs baseline: 1.1606x; 1.0055x over previous
import functools

import jax
import jax.numpy as jnp
from jax import lax
from jax.experimental import pallas as pl
from jax.experimental.pallas import tpu as pltpu
from jax.experimental.pallas import tpu_sc as plsc

B, T, K = 1024, 200, 128
H = K + 5
NC, NS, L = 2, 16, 16
NW = NC * NS
ROWS = B // NW
NBATCH = ROWS // L

_LN2 = 0.6931471805599453


def _log_ge1(x):
    bits = lax.bitcast_convert_type(x, jnp.int32)
    e = (bits >> 23) - 127
    m = lax.bitcast_convert_type((bits & 0x007FFFFF) | 0x3F800000, jnp.float32)
    z = (m - 1.0) / (m + 1.0)
    z2 = z * z
    logm = 2.0 * z * (1.0 + z2 * (1.0 / 3.0 + z2 * (0.2 + z2 * (1.0 / 7.0))))
    return e.astype(jnp.float32) * _LN2 + logm


def _make_agg():
    mesh = plsc.VectorSubcoreMesh(core_axis_name="c", subcore_axis_name="s")

    @functools.partial(
        pl.kernel,
        out_type=jax.ShapeDtypeStruct((B * H,), jnp.float32),
        mesh=mesh,
        compiler_params=pltpu.CompilerParams(needs_layout_passes=False),
        scratch_types=[
            pltpu.VMEM((ROWS * T,), jnp.float32),
            pltpu.VMEM((ROWS * T,), jnp.int32),
            pltpu.VMEM((ROWS,), jnp.int32),
            pltpu.VMEM((ROWS * H,), jnp.float32),
            pltpu.SemaphoreType.DMA,
            pltpu.SemaphoreType.DMA,
            pltpu.SemaphoreType.DMA,
        ],
    )
    def agg(amt_hbm, mcc_hbm, sl_hbm, out_hbm, amt_v, mcc_v, sl_v, outb_v,
            sem_a, sem_m, sem_s):
        wid = lax.axis_index("s") * NC + lax.axis_index("c")
        h_a = pltpu.async_copy(amt_hbm.at[pl.ds(wid * (ROWS * T), ROWS * T)],
                               amt_v, sem_a)
        h_m = pltpu.async_copy(mcc_hbm.at[pl.ds(wid * (ROWS * T), ROWS * T)],
                               mcc_v, sem_m)
        h_s = pltpu.async_copy(sl_hbm.at[pl.ds(wid * ROWS, ROWS)], sl_v, sem_s)

        @plsc.parallel_loop(0, ROWS * H // L, 1, unroll=8)
        def _zero(i):
            outb_v[pl.ds(i * L, L)] = jnp.zeros((L,), jnp.float32)
        h_a.wait()
        h_m.wait()
        h_s.wait()

        lanes = lax.iota(jnp.int32, L)
        ones = jnp.full((L,), 1.0, jnp.float32)
        zeros = jnp.zeros((L,), jnp.float32)

        rowsT = [(lanes + nb * L) * T for nb in range(NBATCH)]
        rowsH = [(lanes + nb * L) * H for nb in range(NBATCH)]

        init = tuple((zeros, zeros, zeros) for _ in range(NBATCH))

        @plsc.parallel_loop(0, T, 1, unroll=4, carry=init)
        def sums(t, carry):
            new = []
            for nb in range(NBATCH):
                sp, sn, st = carry[nb]
                idx = rowsT[nb] + t
                a = plsc.load_gather(amt_v, [idx])
                code = plsc.load_gather(mcc_v, [idx])
                plsc.addupdate_scatter(outb_v, [rowsH[nb] + 4 + code], ones)
                new.append((sp + jnp.maximum(a, 0.0),
                            sn + jnp.minimum(a, 0.0),
                            st + a))
            return tuple(new)

        for nb in range(NBATCH):
            sp, sn, st = sums[nb]
            rH = rowsH[nb]
            sl_f = sl_v[pl.ds(nb * L, L)].astype(jnp.float32)
            plsc.store_scatter(outb_v, [rH], sl_f)
            plsc.store_scatter(outb_v, [rH + 1], _log_ge1(sp + 1.0))
            plsc.store_scatter(outb_v, [rH + 2], -_log_ge1(1.0 - sn))
            plsc.store_scatter(outb_v, [rH + 3], st / (sl_f + 1e-9))
            plsc.store_scatter(outb_v, [rH + 4], zeros)

        @plsc.parallel_loop(0, K - 1, 1, unroll=8,
                            carry=tuple(zeros for _ in range(NBATCH)))
        def distincts(k, accs):
            return tuple(
                accs[nb] + jnp.where(
                    plsc.load_gather(outb_v, [rowsH[nb] + 5 + k]) > 0.0,
                    1.0, 0.0)
                for nb in range(NBATCH))

        for nb in range(NBATCH):
            plsc.store_scatter(outb_v, [rowsH[nb] + (H - 1)], distincts[nb])

        pltpu.sync_copy(outb_v, out_hbm.at[pl.ds(wid * (ROWS * H), ROWS * H)])

    return agg


_agg = _make_agg()


def kernel(amount, mcc_code, seq_lens, ohe_mcc_code):
    del ohe_mcc_code
    out_flat = _agg(amount.reshape(-1), mcc_code.reshape(-1), seq_lens)
    return out_flat.reshape(B, H)
